# baseline (speedup 1.0000x reference)
import functools

import jax
import jax.numpy as jnp
from jax import lax
from jax.experimental import pallas as pl
from jax.experimental.pallas import tpu as pltpu
from jax.experimental.pallas import tpu_sc as plsc

_N = 10000
_E = 320000
_F = 128
_G = 64

_NC = 2
_NS = 16
_NW = _NC * _NS
_CHUNK = 112
_CPW = 90
_CPP = 18
_NCHUNKS = _NW * _CPW
_NPH = _CPW // _CPP
_EPAD = _NCHUNKS * _CHUNK
_NPAD = 10240
_RPS = _NPAD // _NS


def _segment_sum_sc(h, src2d, dst2d, zeros):
    mesh = plsc.VectorSubcoreMesh(core_axis_name="c", subcore_axis_name="s")

    @functools.partial(
        pl.kernel,
        mesh=mesh,
        out_type=jax.ShapeDtypeStruct((_NC, _NPAD, _F), jnp.float32),
        scratch_types=[
            pltpu.VMEM((_CPP, _CHUNK), jnp.int32),
            pltpu.VMEM((_CPP, _CHUNK), jnp.int32),
            pltpu.VMEM((_CHUNK, _F), jnp.float32),
            pltpu.VMEM((_CHUNK, _F), jnp.float32),
            pltpu.VMEM((_CHUNK, _F), jnp.float32),
            pltpu.SemaphoreType.DMA,
            pltpu.SemaphoreType.DMA,
            pltpu.SemaphoreType.DMA,
            pltpu.SemaphoreType.DMA,
            pltpu.SemaphoreType.DMA,
            pltpu.SemaphoreType.DMA,
            pltpu.VMEM_SHARED((_NPAD, _F), jnp.float32),
        ],
    )
    def k(h_hbm, src_hbm, dst_hbm, zeros_hbm, out_hbm,
          src_v, dst_v, b0, b1, b2, sg0, sg1, sg2, ss0, ss1, ss2, acc):
        cid = lax.axis_index("c")
        sid = lax.axis_index("s")
        wid = sid * _NC + cid
        pltpu.sync_copy(zeros_hbm.at[pl.ds(sid * _RPS, _RPS)],
                        acc.at[pl.ds(sid * _RPS, _RPS)])
        plsc.subcore_barrier()

        bufs = (b0, b1, b2)
        sgs = (sg0, sg1, sg2)
        sss = (ss0, ss1, ss2)

        def g_start(c, j):
            pltpu.async_copy(h_hbm.at[src_v.at[c]], bufs[j], sgs[j])

        def g_wait(j):
            pltpu.make_async_copy(h_hbm.at[src_v.at[0]], bufs[j], sgs[j]).wait()

        def s_start(c, j):
            pltpu.async_copy(bufs[j], acc.at[dst_v.at[c]], sss[j], add=True)

        def s_wait(j):
            pltpu.make_async_copy(bufs[j], acc.at[dst_v.at[0]], sss[j]).wait()

        for ph in range(_NPH):
            blk = wid * _NPH + ph
            pltpu.sync_copy(src_hbm.at[blk], src_v)
            pltpu.sync_copy(dst_hbm.at[blk], dst_v)
            g_start(0, 0)
            g_start(1, 1)

            def grp(i, carry):
                c = 3 * i
                g_wait(0)
                s_start(c, 0)

                @pl.when(i > 0)
                def _():
                    s_wait(2)

                g_start(c + 2, 2)
                g_wait(1)
                s_start(c + 1, 1)

                @pl.when(i < _CPP // 3 - 1)
                def _():
                    s_wait(0)
                    g_start(c + 3, 0)

                g_wait(2)
                s_start(c + 2, 2)

                @pl.when(i < _CPP // 3 - 1)
                def _():
                    s_wait(1)
                    g_start(c + 4, 1)

                return carry

            lax.fori_loop(0, _CPP // 3, grp, 0)
            s_wait(0)
            s_wait(1)
            s_wait(2)
        plsc.subcore_barrier()
        pltpu.sync_copy(acc.at[pl.ds(sid * _RPS, _RPS)],
                        out_hbm.at[cid, pl.ds(sid * _RPS, _RPS)])

    return k(h, src2d, dst2d, zeros)


def _gru_block(p_ref, h_ref, wih_ref, whh_ref, bih_ref, bhh_ref):
    m = p_ref[0] + p_ref[1]
    gi = jnp.dot(m, wih_ref[...], preferred_element_type=jnp.float32) + bih_ref[...]
    gh = jnp.dot(h_ref[...], whh_ref[...], preferred_element_type=jnp.float32) + bhh_ref[...]
    r = jax.nn.sigmoid(gi[:, :_F] + gh[:, :_F])
    z = jax.nn.sigmoid(gi[:, _F:2 * _F] + gh[:, _F:2 * _F])
    n = jnp.tanh(gi[:, 2 * _F:] + r * gh[:, 2 * _F:])
    return (1.0 - z) * n + z * h_ref[...]


_R = 1000


def _gru_tc(p, h, wihT, whhT, bih, bhh):
    def body(p_ref, h_ref, wih_ref, whh_ref, bih_ref, bhh_ref, out_ref):
        out_ref[...] = _gru_block(p_ref, h_ref, wih_ref, whh_ref, bih_ref, bhh_ref)

    return pl.pallas_call(
        body,
        grid=(_N // _R,),
        in_specs=[
            pl.BlockSpec((2, _R, _F), lambda i: (0, i, 0)),
            pl.BlockSpec((_R, _F), lambda i: (i, 0)),
            pl.BlockSpec((_F, 3 * _F), lambda i: (0, 0)),
            pl.BlockSpec((_F, 3 * _F), lambda i: (0, 0)),
            pl.BlockSpec((1, 3 * _F), lambda i: (0, 0)),
            pl.BlockSpec((1, 3 * _F), lambda i: (0, 0)),
        ],
        out_specs=pl.BlockSpec((_R, _F), lambda i: (i, 0)),
        out_shape=jax.ShapeDtypeStruct((_N, _F), jnp.float32),
    )(p, h, wihT, whhT, bih, bhh)


def _softsign(x):
    return x / (1.0 + jnp.abs(x))


def _gru_attn_tc(q, h, data, wihT, whhT, bih, bhh, wi1h, wi1d, bi1v, wi2T, bi2v, wjT, bjv):
    def body(q_ref, h_ref, d_ref, wih_ref, whh_ref, bih_ref, bhh_ref,
             wi1h_ref, wi1d_ref, bi1_ref, wi2_ref, bi2_ref, wj_ref, bj_ref, out_ref):
        h2 = _gru_block(q_ref, h_ref, wih_ref, whh_ref, bih_ref, bhh_ref)
        d = d_ref[...]
        a = _softsign(jnp.dot(h2, wi1h_ref[...], preferred_element_type=jnp.float32)
                      + jnp.dot(d, wi1d_ref[...], preferred_element_type=jnp.float32)
                      + bi1_ref[...])
        a = _softsign(jnp.dot(a, wi2_ref[...], preferred_element_type=jnp.float32)
                      + bi2_ref[...])
        a = a - jnp.max(a, axis=1, keepdims=True)
        a = jnp.exp(a)
        a = a / jnp.sum(a, axis=1, keepdims=True)
        j = _softsign(jnp.dot(d, wj_ref[...], preferred_element_type=jnp.float32)
                      + bj_ref[...])
        out_ref[...] = a * j

    return pl.pallas_call(
        body,
        grid=(_N // _R,),
        in_specs=[
            pl.BlockSpec((2, _R, _F), lambda i: (0, i, 0)),
            pl.BlockSpec((_R, _F), lambda i: (i, 0)),
            pl.BlockSpec((_R, _F), lambda i: (i, 0)),
            pl.BlockSpec((_F, 3 * _F), lambda i: (0, 0)),
            pl.BlockSpec((_F, 3 * _F), lambda i: (0, 0)),
            pl.BlockSpec((1, 3 * _F), lambda i: (0, 0)),
            pl.BlockSpec((1, 3 * _F), lambda i: (0, 0)),
            pl.BlockSpec((_F, _F), lambda i: (0, 0)),
            pl.BlockSpec((_F, _F), lambda i: (0, 0)),
            pl.BlockSpec((1, _F), lambda i: (0, 0)),
            pl.BlockSpec((_F, _G), lambda i: (0, 0)),
            pl.BlockSpec((1, _G), lambda i: (0, 0)),
            pl.BlockSpec((_F, _G), lambda i: (0, 0)),
            pl.BlockSpec((1, _G), lambda i: (0, 0)),
        ],
        out_specs=pl.BlockSpec((_R, _G), lambda i: (i, 0)),
        out_shape=jax.ShapeDtypeStruct((_N, _G), jnp.float32),
    )(q, h, data, wihT, whhT, bih, bhh, wi1h, wi1d, bi1v, wi2T, bi2v, wjT, bjv)


def kernel(data, edge_index, edge_attr, weight, w_ih, w_hh, b_ih, b_hh, wi1, bi1, wi2, bi2, wj, bj):
    del edge_attr, weight
    npad = _EPAD - _E
    src2d = jnp.concatenate(
        [edge_index[0], jnp.zeros((npad,), jnp.int32)]
    ).reshape(_NW * _NPH, _CPP, _CHUNK)
    dst2d = jnp.concatenate(
        [edge_index[1], jnp.full((npad,), _NPAD - 1, jnp.int32)]
    ).reshape(_NW * _NPH, _CPP, _CHUNK)
    zeros = jnp.zeros((_NPAD, _F), jnp.float32)

    wihT = w_ih.T
    whhT = w_hh.T
    bihv = b_ih.reshape(1, -1)
    bhhv = b_hh.reshape(1, -1)
    wi1T = wi1.T
    wi1h = wi1T[:_F]
    wi1d = wi1T[_F:]
    bi1v = bi1.reshape(1, -1)
    wi2T = wi2.T
    bi2v = bi2.reshape(1, -1)
    wjT = wj.T
    bjv = bj.reshape(1, -1)

    p = _segment_sum_sc(data, src2d, dst2d, zeros)
    h1 = _gru_tc(p, data, wihT, whhT, bihv, bhhv)
    q = _segment_sum_sc(h1, src2d, dst2d, zeros)
    return _gru_attn_tc(q, h1, data, wihT, whhT, bihv, bhhv,
                        wi1h, wi1d, bi1v, wi2T, bi2v, wjT, bjv)


# speedup vs baseline: 1.5731x; 1.5731x over previous
import functools

import jax
import jax.numpy as jnp
from jax import lax
from jax.experimental import pallas as pl
from jax.experimental.pallas import tpu as pltpu
from jax.experimental.pallas import tpu_sc as plsc

_N = 10000
_E = 320000
_F = 128
_G = 64

_NC = 2
_NS = 16
_NW = _NC * _NS
_CHUNK = 125
_CPW = 80
_CPP = 40
_NCHUNKS = _NW * _CPW
_NPH = _CPW // _CPP
_EPAD = _NCHUNKS * _CHUNK
_NPAD = 10240
_RPS = _NPAD // _NS


def _segment_sum_sc(h, src2d, dst2d, zeros):
    mesh = plsc.VectorSubcoreMesh(core_axis_name="c", subcore_axis_name="s")

    @functools.partial(
        pl.kernel,
        mesh=mesh,
        out_type=jax.ShapeDtypeStruct((_NC, _NPAD, _F), jnp.float32),
        scratch_types=[
            pltpu.VMEM((_CPP, _CHUNK), jnp.int32),
            pltpu.VMEM((_CPP, _CHUNK), jnp.int32),
            pltpu.VMEM((_CHUNK, _F), jnp.float32),
            pltpu.VMEM((_CHUNK, _F), jnp.float32),
            pltpu.SemaphoreType.DMA,
            pltpu.SemaphoreType.DMA,
            pltpu.VMEM_SHARED((_NPAD, _F), jnp.float32),
        ],
    )
    def k(h_hbm, src_hbm, dst_hbm, zeros_hbm, out_hbm,
          src_v, dst_v, rows0, rows1, sem0, sem1, acc):
        cid = lax.axis_index("c")
        sid = lax.axis_index("s")
        wid = sid * _NC + cid
        pltpu.sync_copy(zeros_hbm.at[pl.ds(sid * _RPS, _RPS)],
                        acc.at[pl.ds(sid * _RPS, _RPS)])
        plsc.subcore_barrier()

        for ph in range(_NPH):
            blk = wid * _NPH + ph
            pltpu.sync_copy(src_hbm.at[blk], src_v)
            pltpu.sync_copy(dst_hbm.at[blk], dst_v)
            pltpu.async_copy(h_hbm.at[src_v.at[0]], rows0, sem0)

            def body(i, carry):
                c = 2 * i
                pltpu.make_async_copy(h_hbm.at[src_v.at[c]], rows0, sem0).wait()
                pltpu.async_copy(h_hbm.at[src_v.at[c + 1]], rows1, sem1)
                pltpu.sync_copy(rows0, acc.at[dst_v.at[c]], add=True)
                pltpu.make_async_copy(h_hbm.at[src_v.at[c + 1]], rows1, sem1).wait()

                @pl.when(c + 2 < _CPP)
                def _():
                    pltpu.async_copy(h_hbm.at[src_v.at[c + 2]], rows0, sem0)

                pltpu.sync_copy(rows1, acc.at[dst_v.at[c + 1]], add=True)
                return carry

            lax.fori_loop(0, _CPP // 2, body, 0)
        plsc.subcore_barrier()
        pltpu.sync_copy(acc.at[pl.ds(sid * _RPS, _RPS)],
                        out_hbm.at[cid, pl.ds(sid * _RPS, _RPS)])

    return k(h, src2d, dst2d, zeros)


def _gru_block(p_ref, h_ref, wih_ref, whh_ref, bih_ref, bhh_ref):
    m = p_ref[0] + p_ref[1]
    gi = jnp.dot(m, wih_ref[...], preferred_element_type=jnp.float32) + bih_ref[...]
    gh = jnp.dot(h_ref[...], whh_ref[...], preferred_element_type=jnp.float32) + bhh_ref[...]
    r = jax.nn.sigmoid(gi[:, :_F] + gh[:, :_F])
    z = jax.nn.sigmoid(gi[:, _F:2 * _F] + gh[:, _F:2 * _F])
    n = jnp.tanh(gi[:, 2 * _F:] + r * gh[:, 2 * _F:])
    return (1.0 - z) * n + z * h_ref[...]


_R = 1000


def _gru_tc(p, h, wihT, whhT, bih, bhh):
    def body(p_ref, h_ref, wih_ref, whh_ref, bih_ref, bhh_ref, out_ref):
        out_ref[...] = _gru_block(p_ref, h_ref, wih_ref, whh_ref, bih_ref, bhh_ref)

    return pl.pallas_call(
        body,
        grid=(_N // _R,),
        in_specs=[
            pl.BlockSpec((2, _R, _F), lambda i: (0, i, 0)),
            pl.BlockSpec((_R, _F), lambda i: (i, 0)),
            pl.BlockSpec((_F, 3 * _F), lambda i: (0, 0)),
            pl.BlockSpec((_F, 3 * _F), lambda i: (0, 0)),
            pl.BlockSpec((1, 3 * _F), lambda i: (0, 0)),
            pl.BlockSpec((1, 3 * _F), lambda i: (0, 0)),
        ],
        out_specs=pl.BlockSpec((_R, _F), lambda i: (i, 0)),
        out_shape=jax.ShapeDtypeStruct((_N, _F), jnp.float32),
    )(p, h, wihT, whhT, bih, bhh)


def _softsign(x):
    return x / (1.0 + jnp.abs(x))


def _gru_attn_tc(q, h, data, wihT, whhT, bih, bhh, wi1h, wi1d, bi1v, wi2T, bi2v, wjT, bjv):
    def body(q_ref, h_ref, d_ref, wih_ref, whh_ref, bih_ref, bhh_ref,
             wi1h_ref, wi1d_ref, bi1_ref, wi2_ref, bi2_ref, wj_ref, bj_ref, out_ref):
        h2 = _gru_block(q_ref, h_ref, wih_ref, whh_ref, bih_ref, bhh_ref)
        d = d_ref[...]
        a = _softsign(jnp.dot(h2, wi1h_ref[...], preferred_element_type=jnp.float32)
                      + jnp.dot(d, wi1d_ref[...], preferred_element_type=jnp.float32)
                      + bi1_ref[...])
        a = _softsign(jnp.dot(a, wi2_ref[...], preferred_element_type=jnp.float32)
                      + bi2_ref[...])
        a = a - jnp.max(a, axis=1, keepdims=True)
        a = jnp.exp(a)
        a = a / jnp.sum(a, axis=1, keepdims=True)
        j = _softsign(jnp.dot(d, wj_ref[...], preferred_element_type=jnp.float32)
                      + bj_ref[...])
        out_ref[...] = a * j

    return pl.pallas_call(
        body,
        grid=(_N // _R,),
        in_specs=[
            pl.BlockSpec((2, _R, _F), lambda i: (0, i, 0)),
            pl.BlockSpec((_R, _F), lambda i: (i, 0)),
            pl.BlockSpec((_R, _F), lambda i: (i, 0)),
            pl.BlockSpec((_F, 3 * _F), lambda i: (0, 0)),
            pl.BlockSpec((_F, 3 * _F), lambda i: (0, 0)),
            pl.BlockSpec((1, 3 * _F), lambda i: (0, 0)),
            pl.BlockSpec((1, 3 * _F), lambda i: (0, 0)),
            pl.BlockSpec((_F, _F), lambda i: (0, 0)),
            pl.BlockSpec((_F, _F), lambda i: (0, 0)),
            pl.BlockSpec((1, _F), lambda i: (0, 0)),
            pl.BlockSpec((_F, _G), lambda i: (0, 0)),
            pl.BlockSpec((1, _G), lambda i: (0, 0)),
            pl.BlockSpec((_F, _G), lambda i: (0, 0)),
            pl.BlockSpec((1, _G), lambda i: (0, 0)),
        ],
        out_specs=pl.BlockSpec((_R, _G), lambda i: (i, 0)),
        out_shape=jax.ShapeDtypeStruct((_N, _G), jnp.float32),
    )(q, h, data, wihT, whhT, bih, bhh, wi1h, wi1d, bi1v, wi2T, bi2v, wjT, bjv)


def kernel(data, edge_index, edge_attr, weight, w_ih, w_hh, b_ih, b_hh, wi1, bi1, wi2, bi2, wj, bj):
    del edge_attr, weight
    src2d = edge_index[0].reshape(_NW * _NPH, _CPP, _CHUNK)
    dst2d = edge_index[1].reshape(_NW * _NPH, _CPP, _CHUNK)
    zeros = jnp.zeros((_NPAD, _F), jnp.float32)

    wihT = w_ih.T
    whhT = w_hh.T
    bihv = b_ih.reshape(1, -1)
    bhhv = b_hh.reshape(1, -1)
    wi1T = wi1.T
    wi1h = wi1T[:_F]
    wi1d = wi1T[_F:]
    bi1v = bi1.reshape(1, -1)
    wi2T = wi2.T
    bi2v = bi2.reshape(1, -1)
    wjT = wj.T
    bjv = bj.reshape(1, -1)

    p = _segment_sum_sc(data, src2d, dst2d, zeros)
    h1 = _gru_tc(p, data, wihT, whhT, bihv, bhhv)
    q = _segment_sum_sc(h1, src2d, dst2d, zeros)
    return _gru_attn_tc(q, h1, data, wihT, whhT, bihv, bhhv,
                        wi1h, wi1d, bi1v, wi2T, bi2v, wjT, bjv)


# speedup vs baseline: 1.6071x; 1.0216x over previous
import functools

import jax
import jax.numpy as jnp
from jax import lax
from jax.experimental import pallas as pl
from jax.experimental.pallas import tpu as pltpu
from jax.experimental.pallas import tpu_sc as plsc

_N = 10000
_E = 320000
_F = 128
_G = 64

_NC = 2
_NS = 16
_NW = _NC * _NS
_CHUNK = 125
_CPW = 80
_CPP = 40
_NCHUNKS = _NW * _CPW
_NPH = _CPW // _CPP
_EPAD = _NCHUNKS * _CHUNK
_NPAD = 10240
_RPS = _NPAD // _NS


def _segment_sum_sc(h, src2d, dst2d, zeros):
    mesh = plsc.VectorSubcoreMesh(core_axis_name="c", subcore_axis_name="s")

    @functools.partial(
        pl.kernel,
        mesh=mesh,
        out_type=jax.ShapeDtypeStruct((_NC, _NPAD, _F), jnp.float32),
        scratch_types=[
            pltpu.VMEM((_CPP, _CHUNK), jnp.int32),
            pltpu.VMEM((_CPP, _CHUNK), jnp.int32),
            pltpu.VMEM((_CHUNK, _F), jnp.float32),
            pltpu.VMEM((_CHUNK, _F), jnp.float32),
            pltpu.SemaphoreType.DMA,
            pltpu.SemaphoreType.DMA,
            pltpu.VMEM_SHARED((_NPAD, _F), jnp.float32),
        ],
    )
    def k(h_hbm, src_hbm, dst_hbm, zeros_hbm, out_hbm,
          src_v, dst_v, rows0, rows1, sem0, sem1, acc):
        cid = lax.axis_index("c")
        sid = lax.axis_index("s")
        wid = sid * _NC + cid
        pltpu.sync_copy(zeros_hbm.at[pl.ds(sid * _RPS, _RPS)],
                        acc.at[pl.ds(sid * _RPS, _RPS)])
        plsc.subcore_barrier()

        for ph in range(_NPH):
            blk = wid * _NPH + ph
            pltpu.sync_copy(src_hbm.at[blk], src_v)
            pltpu.sync_copy(dst_hbm.at[blk], dst_v)
            pltpu.async_copy(h_hbm.at[src_v.at[0]], rows0, sem0)

            def body(i, carry):
                c = 2 * i
                pltpu.make_async_copy(h_hbm.at[src_v.at[c]], rows0, sem0).wait()
                pltpu.async_copy(h_hbm.at[src_v.at[c + 1]], rows1, sem1)
                pltpu.sync_copy(rows0, acc.at[dst_v.at[c]], add=True)
                pltpu.make_async_copy(h_hbm.at[src_v.at[c + 1]], rows1, sem1).wait()

                @pl.when(c + 2 < _CPP)
                def _():
                    pltpu.async_copy(h_hbm.at[src_v.at[c + 2]], rows0, sem0)

                pltpu.sync_copy(rows1, acc.at[dst_v.at[c + 1]], add=True)
                return carry

            lax.fori_loop(0, _CPP // 2, body, 0)
        plsc.subcore_barrier()
        pltpu.sync_copy(acc.at[pl.ds(sid * _RPS, _RPS)],
                        out_hbm.at[cid, pl.ds(sid * _RPS, _RPS)])

    return k(h, src2d, dst2d, zeros)


def _gru_block(p_ref, h_ref, wih_ref, whh_ref, bih_ref, bhh_ref):
    m = p_ref[0] + p_ref[1]
    gi = jnp.dot(m, wih_ref[...], preferred_element_type=jnp.float32) + bih_ref[...]
    gh = jnp.dot(h_ref[...], whh_ref[...], preferred_element_type=jnp.float32) + bhh_ref[...]
    r = jax.nn.sigmoid(gi[:, :_F] + gh[:, :_F])
    z = jax.nn.sigmoid(gi[:, _F:2 * _F] + gh[:, _F:2 * _F])
    n = jnp.tanh(gi[:, 2 * _F:] + r * gh[:, 2 * _F:])
    return (1.0 - z) * n + z * h_ref[...]


_R = 2000


def _gru_tc(p, h, wihT, whhT, bih, bhh):
    def body(p_ref, h_ref, wih_ref, whh_ref, bih_ref, bhh_ref, out_ref):
        out_ref[...] = _gru_block(p_ref, h_ref, wih_ref, whh_ref, bih_ref, bhh_ref)

    return pl.pallas_call(
        body,
        grid=(_N // _R,),
        in_specs=[
            pl.BlockSpec((2, _R, _F), lambda i: (0, i, 0)),
            pl.BlockSpec((_R, _F), lambda i: (i, 0)),
            pl.BlockSpec((_F, 3 * _F), lambda i: (0, 0)),
            pl.BlockSpec((_F, 3 * _F), lambda i: (0, 0)),
            pl.BlockSpec((1, 3 * _F), lambda i: (0, 0)),
            pl.BlockSpec((1, 3 * _F), lambda i: (0, 0)),
        ],
        out_specs=pl.BlockSpec((_R, _F), lambda i: (i, 0)),
        out_shape=jax.ShapeDtypeStruct((_N, _F), jnp.float32),
    )(p, h, wihT, whhT, bih, bhh)


def _softsign(x):
    return x / (1.0 + jnp.abs(x))


def _gru_attn_tc(q, h, data, wihT, whhT, bih, bhh, wi1h, wi1d, bi1v, wi2T, bi2v, wjT, bjv):
    def body(q_ref, h_ref, d_ref, wih_ref, whh_ref, bih_ref, bhh_ref,
             wi1h_ref, wi1d_ref, bi1_ref, wi2_ref, bi2_ref, wj_ref, bj_ref, out_ref):
        h2 = _gru_block(q_ref, h_ref, wih_ref, whh_ref, bih_ref, bhh_ref)
        d = d_ref[...]
        a = _softsign(jnp.dot(h2, wi1h_ref[...], preferred_element_type=jnp.float32)
                      + jnp.dot(d, wi1d_ref[...], preferred_element_type=jnp.float32)
                      + bi1_ref[...])
        a = _softsign(jnp.dot(a, wi2_ref[...], preferred_element_type=jnp.float32)
                      + bi2_ref[...])
        a = a - jnp.max(a, axis=1, keepdims=True)
        a = jnp.exp(a)
        a = a / jnp.sum(a, axis=1, keepdims=True)
        j = _softsign(jnp.dot(d, wj_ref[...], preferred_element_type=jnp.float32)
                      + bj_ref[...])
        out_ref[...] = a * j

    return pl.pallas_call(
        body,
        grid=(_N // _R,),
        in_specs=[
            pl.BlockSpec((2, _R, _F), lambda i: (0, i, 0)),
            pl.BlockSpec((_R, _F), lambda i: (i, 0)),
            pl.BlockSpec((_R, _F), lambda i: (i, 0)),
            pl.BlockSpec((_F, 3 * _F), lambda i: (0, 0)),
            pl.BlockSpec((_F, 3 * _F), lambda i: (0, 0)),
            pl.BlockSpec((1, 3 * _F), lambda i: (0, 0)),
            pl.BlockSpec((1, 3 * _F), lambda i: (0, 0)),
            pl.BlockSpec((_F, _F), lambda i: (0, 0)),
            pl.BlockSpec((_F, _F), lambda i: (0, 0)),
            pl.BlockSpec((1, _F), lambda i: (0, 0)),
            pl.BlockSpec((_F, _G), lambda i: (0, 0)),
            pl.BlockSpec((1, _G), lambda i: (0, 0)),
            pl.BlockSpec((_F, _G), lambda i: (0, 0)),
            pl.BlockSpec((1, _G), lambda i: (0, 0)),
        ],
        out_specs=pl.BlockSpec((_R, _G), lambda i: (i, 0)),
        out_shape=jax.ShapeDtypeStruct((_N, _G), jnp.float32),
    )(q, h, data, wihT, whhT, bih, bhh, wi1h, wi1d, bi1v, wi2T, bi2v, wjT, bjv)


def kernel(data, edge_index, edge_attr, weight, w_ih, w_hh, b_ih, b_hh, wi1, bi1, wi2, bi2, wj, bj):
    del edge_attr, weight
    src2d = edge_index[0].reshape(_NW * _NPH, _CPP, _CHUNK)
    dst2d = edge_index[1].reshape(_NW * _NPH, _CPP, _CHUNK)
    zeros = jnp.zeros((_NPAD, _F), jnp.float32)

    wihT = w_ih.T
    whhT = w_hh.T
    bihv = b_ih.reshape(1, -1)
    bhhv = b_hh.reshape(1, -1)
    wi1T = wi1.T
    wi1h = wi1T[:_F]
    wi1d = wi1T[_F:]
    bi1v = bi1.reshape(1, -1)
    wi2T = wi2.T
    bi2v = bi2.reshape(1, -1)
    wjT = wj.T
    bjv = bj.reshape(1, -1)

    p = _segment_sum_sc(data, src2d, dst2d, zeros)
    h1 = _gru_tc(p, data, wihT, whhT, bihv, bhhv)
    q = _segment_sum_sc(h1, src2d, dst2d, zeros)
    return _gru_attn_tc(q, h1, data, wihT, whhT, bihv, bhhv,
                        wi1h, wi1d, bi1v, wi2T, bi2v, wjT, bjv)
